# baseline (speedup 1.0000x reference)
import functools

import jax
import jax.numpy as jnp
from jax import lax
from jax.experimental import pallas as pl
from jax.experimental.pallas import tpu as pltpu
from jax.experimental.pallas import tpu_sc as plsc

BATCH = 16384
D = 64
CHUNK = 128
_info = plsc.get_sparse_core_info()
NC, NS = _info.num_cores, _info.num_subcores
NW = NC * NS
B_PER_W = BATCH // NW
CHUNKS_PER_W = B_PER_W // CHUNK


def _sc_gather_kernel(user_hbm, item_hbm, mfu_hbm, mfi_hbm, mlu_hbm, mli_hbm,
                      out_mfu, out_mfi, out_mlu, out_mli,
                      idx_u, idx_i, rows_a, rows_b, sem_a, sem_b):
    wid = lax.axis_index("s") * NC + lax.axis_index("c")
    base = wid * B_PER_W
    pltpu.sync_copy(user_hbm.at[pl.ds(wid * CHUNKS_PER_W, CHUNKS_PER_W)], idx_u)
    pltpu.sync_copy(item_hbm.at[pl.ds(wid * CHUNKS_PER_W, CHUNKS_PER_W)], idx_i)

    for t, (table, idx, out) in enumerate((
            (mfu_hbm, idx_u, out_mfu),
            (mfi_hbm, idx_i, out_mfi),
            (mlu_hbm, idx_u, out_mlu),
            (mli_hbm, idx_i, out_mli))):
        for j in range(CHUNKS_PER_W):
            rows, sem = (rows_a, sem_a) if (t * CHUNKS_PER_W + j) % 2 == 0 else (rows_b, sem_b)
            pltpu.async_copy(table.at[idx.at[j]], rows, sem).wait()
            pltpu.sync_copy(rows, out.at[pl.ds(base + j * CHUNK, CHUNK)])


def _sc_gather(user, item, mfu, mfi, mlu, mli):
    mesh = plsc.VectorSubcoreMesh(core_axis_name="c", subcore_axis_name="s")
    f32 = jnp.float32
    out_type = [jax.ShapeDtypeStruct((BATCH, D), f32)] * 4
    kern = pl.kernel(
        _sc_gather_kernel,
        mesh=mesh,
        out_type=out_type,
        compiler_params=pltpu.CompilerParams(use_tc_tiling_on_sc=False),
        scratch_types=[
            pltpu.VMEM((CHUNKS_PER_W, CHUNK), jnp.int32),
            pltpu.VMEM((CHUNKS_PER_W, CHUNK), jnp.int32),
            pltpu.VMEM((CHUNK, D), f32),
            pltpu.VMEM((CHUNK, D), f32),
            pltpu.SemaphoreType.DMA,
            pltpu.SemaphoreType.DMA,
        ],
    )
    u2 = user.reshape(BATCH // CHUNK, CHUNK)
    i2 = item.reshape(BATCH // CHUNK, CHUNK)
    return kern(u2, i2, mfu, mfi, mlu, mli)


def _tc_dense_kernel(mfu_ref, mfi_ref, mlu_ref, mli_ref,
                     W0_ref, b0_ref, W1_ref, b1_ref, W2_ref, b2_ref,
                     Wo_ref, bo_ref, out_ref):
    mf = mfu_ref[...] * mfi_ref[...]
    W0 = W0_ref[...]
    h = mlu_ref[...] @ W0[:D, :] + mli_ref[...] @ W0[D:, :] + b0_ref[...]
    h = jnp.maximum(h, 0.0)
    h = jnp.maximum(h @ W1_ref[...] + b1_ref[...], 0.0)
    h = jnp.maximum(h @ W2_ref[...] + b2_ref[...], 0.0)
    Wo = Wo_ref[...]
    out = mf @ Wo[:D, :] + h @ Wo[D:, :] + bo_ref[...]
    out_ref[...] = out


def _tc_dense(mfu, mfi, mlu, mli, W0, b0, W1, b1, W2, b2, Wo, bo):
    BLK = 2048
    grid = (BATCH // BLK,)
    row_spec = pl.BlockSpec((BLK, D), lambda i: (i, 0))
    full = lambda shape: pl.BlockSpec(shape, lambda i: tuple(0 for _ in shape))
    return pl.pallas_call(
        _tc_dense_kernel,
        grid=grid,
        in_specs=[
            row_spec, row_spec, row_spec, row_spec,
            full(W0.shape), full(b0.shape), full(W1.shape), full(b1.shape),
            full(W2.shape), full(b2.shape), full(Wo.shape), full(bo.shape),
        ],
        out_specs=pl.BlockSpec((BLK, 1), lambda i: (i, 0)),
        out_shape=jax.ShapeDtypeStruct((BATCH, 1), jnp.float32),
    )(mfu, mfi, mlu, mli, W0, b0, W1, b1, W2, b2, Wo, bo)


def kernel(user, item, mf_user_emb, mf_item_emb, mlp_user_emb, mlp_item_emb,
           W0, b0, W1, b1, W2, b2, Wo, bo):
    mfu, mfi, mlu, mli = _sc_gather(user, item, mf_user_emb, mf_item_emb,
                                    mlp_user_emb, mlp_item_emb)
    return _tc_dense(mfu, mfi, mlu, mli, W0, b0, W1, b1, W2, b2, Wo, bo)


# speedup vs baseline: 1.7761x; 1.7761x over previous
import functools

import jax
import jax.numpy as jnp
from jax import lax
from jax.experimental import pallas as pl
from jax.experimental.pallas import tpu as pltpu
from jax.experimental.pallas import tpu_sc as plsc

BATCH = 16384
D = 64
NROWS = 100000
CHUNK = 128
_info = plsc.get_sparse_core_info()
NC, NS = _info.num_cores, _info.num_subcores
NW = NC * NS
B_PER_W = BATCH // NW
CHUNKS_PER_W = B_PER_W // CHUNK


def _tc_pack_kernel(a_ref, b_ref, out_ref):
    out_ref[...] = jnp.concatenate(
        [a_ref[...].T, b_ref[...].T], axis=1)


def _tc_pack(a, b):
    CB = 2560
    grid = ((NROWS + CB - 1) // CB,)
    in_spec = pl.BlockSpec((D, CB), lambda i: (0, i))
    return pl.pallas_call(
        _tc_pack_kernel,
        grid=grid,
        in_specs=[in_spec, in_spec],
        out_specs=pl.BlockSpec((CB, 2 * D), lambda i: (i, 0)),
        out_shape=jax.ShapeDtypeStruct((NROWS, 2 * D), jnp.float32),
    )(a.T, b.T)


def _sc_gather_kernel(idx_hbm, tab_hbm, out_hbm, idx_v, rows_a, rows_b,
                      sem_a, sem_b):
    wid = lax.axis_index("s") * NC + lax.axis_index("c")
    base = wid * B_PER_W
    pltpu.sync_copy(idx_hbm.at[pl.ds(wid * CHUNKS_PER_W, CHUNKS_PER_W)], idx_v)
    for j in range(CHUNKS_PER_W):
        rows, sem = (rows_a, sem_a) if j % 2 == 0 else (rows_b, sem_b)
        pltpu.async_copy(tab_hbm.at[idx_v.at[j]], rows, sem).wait()
        pltpu.sync_copy(rows, out_hbm.at[pl.ds(base + j * CHUNK, CHUNK)])


def _sc_gather(idx, tab):
    mesh = plsc.VectorSubcoreMesh(core_axis_name="c", subcore_axis_name="s")
    kern = pl.kernel(
        _sc_gather_kernel,
        mesh=mesh,
        out_type=jax.ShapeDtypeStruct((BATCH, 2 * D), jnp.float32),
        scratch_types=[
            pltpu.VMEM((CHUNKS_PER_W, CHUNK), jnp.int32),
            pltpu.VMEM((CHUNK, 2 * D), jnp.float32),
            pltpu.VMEM((CHUNK, 2 * D), jnp.float32),
            pltpu.SemaphoreType.DMA,
            pltpu.SemaphoreType.DMA,
        ],
    )
    return kern(idx.reshape(BATCH // CHUNK, CHUNK), tab)


def _tc_dense_kernel(gu_ref, gi_ref,
                     W0_ref, b0_ref, W1_ref, b1_ref, W2_ref, b2_ref,
                     Wo_ref, bo_ref, out_ref):
    gu = gu_ref[...]
    gi = gi_ref[...]
    mf = gu[:, :D] * gi[:, :D]
    W0 = W0_ref[...]
    h = gu[:, D:] @ W0[:D, :] + gi[:, D:] @ W0[D:, :] + b0_ref[...]
    h = jnp.maximum(h, 0.0)
    h = jnp.maximum(h @ W1_ref[...] + b1_ref[...], 0.0)
    h = jnp.maximum(h @ W2_ref[...] + b2_ref[...], 0.0)
    Wo = Wo_ref[...]
    out_ref[...] = mf @ Wo[:D, :] + h @ Wo[D:, :] + bo_ref[...]


def _tc_dense(gu, gi, W0, b0, W1, b1, W2, b2, Wo, bo):
    BLK = 2048
    grid = (BATCH // BLK,)
    row_spec = pl.BlockSpec((BLK, 2 * D), lambda i: (i, 0))
    full = lambda shape: pl.BlockSpec(shape, lambda i: tuple(0 for _ in shape))
    return pl.pallas_call(
        _tc_dense_kernel,
        grid=grid,
        in_specs=[
            row_spec, row_spec,
            full(W0.shape), full(b0.shape), full(W1.shape), full(b1.shape),
            full(W2.shape), full(b2.shape), full(Wo.shape), full(bo.shape),
        ],
        out_specs=pl.BlockSpec((BLK, 1), lambda i: (i, 0)),
        out_shape=jax.ShapeDtypeStruct((BATCH, 1), jnp.float32),
    )(gu, gi, W0, b0, W1, b1, W2, b2, Wo, bo)


def kernel(user, item, mf_user_emb, mf_item_emb, mlp_user_emb, mlp_item_emb,
           W0, b0, W1, b1, W2, b2, Wo, bo):
    tab_u = _tc_pack(mf_user_emb, mlp_user_emb)
    gu = _sc_gather(user, tab_u)
    tab_i = _tc_pack(mf_item_emb, mlp_item_emb)
    gi = _sc_gather(item, tab_i)
    return _tc_dense(gu, gi, W0, b0, W1, b1, W2, b2, Wo, bo)
